# baseline (speedup 1.0000x reference)
import functools

import jax
import jax.numpy as jnp
from jax import lax
from jax.experimental import pallas as pl
from jax.experimental.pallas import tpu as pltpu
from jax.experimental.pallas import tpu_sc as plsc

N = 128
E = 1024
_NC = 2
_NS = 16
_EPW = E // _NS


def _adj_sc_body(ei_hbm, out_hbm, zero_v, ones_v, src_v, dst_v, idx_v, acc_sh):
    cid = lax.axis_index("c")
    sid = lax.axis_index("s")
    base = sid * _EPW
    stripe = N * N // _NS

    for i in range(0, stripe, 16):
        zero_v[pl.ds(i, 16)] = jnp.zeros((16,), jnp.float32)
    for i in range(0, _EPW, 16):
        ones_v[pl.ds(i, 16)] = jnp.ones((16,), jnp.float32)

    pltpu.sync_copy(zero_v, acc_sh.at[pl.ds(sid * stripe, stripe)])

    pltpu.sync_copy(ei_hbm.at[cid, pl.ds(base, _EPW)], src_v)
    pltpu.sync_copy(ei_hbm.at[cid, pl.ds(E + base, _EPW)], dst_v)
    for i in range(0, _EPW, 16):
        idx_v[pl.ds(i, 16)] = dst_v[pl.ds(i, 16)] * N + src_v[pl.ds(i, 16)]

    plsc.subcore_barrier()
    pltpu.sync_copy(ones_v, acc_sh.at[idx_v], add=True)
    plsc.subcore_barrier()

    pltpu.sync_copy(acc_sh.at[pl.ds(sid * stripe, stripe)],
                    out_hbm.at[cid, pl.ds(sid * stripe, stripe)])


@functools.cache
def _get_adjacency_sc():
    return pl.kernel(
        _adj_sc_body,
        out_type=jax.ShapeDtypeStruct((_NC, N * N), jnp.float32),
        mesh=plsc.VectorSubcoreMesh(core_axis_name="c", subcore_axis_name="s",
                                    num_cores=_NC, num_subcores=_NS),
        scratch_types=[
            pltpu.VMEM((N * N // _NS,), jnp.float32),
            pltpu.VMEM((_EPW,), jnp.float32),
            pltpu.VMEM((_EPW,), jnp.int32),
            pltpu.VMEM((_EPW,), jnp.int32),
            pltpu.VMEM((_EPW,), jnp.int32),
            pltpu.VMEM_SHARED((N * N,), jnp.float32),
        ],
    )


def _r16(x):
    return x.astype(jnp.bfloat16).astype(jnp.float32)


def _mm(a, b):
    return jnp.dot(a, b, preferred_element_type=jnp.float32)


def _mm_hi(a, b):
    return jnp.dot(a, b, preferred_element_type=jnp.float32,
                   precision=jax.lax.Precision.HIGHEST)


def _bn(h, g, bt):
    m = jnp.mean(h, axis=0, keepdims=True)
    v = jnp.mean((h - m) ** 2, axis=0, keepdims=True)
    return (h - m) / jnp.sqrt(v + 1e-5) * g + bt


def _gin(x, A, eps, W1, b1, W2, b2, g, bt):
    z = (1.0 + eps) * x + jnp.dot(A, x, preferred_element_type=jnp.float32,
                                  precision=jax.lax.Precision.HIGHEST)
    h = jax.nn.relu(_mm(z, W1) + b1)
    h = _mm(h, W2) + b2
    return _bn(h, g, bt)


def _conv_pass(x, A, eps, p):
    h = jax.nn.relu(_gin(x, A, eps[0, 0], p['c1_W1'], p['c1_b1'], p['c1_W2'],
                         p['c1_b2'], p['c1_g'], p['c1_bt']))
    h = jax.nn.relu(_gin(h, A, eps[0, 1], p['c2_W1'], p['c2_b1'], p['c2_W2'],
                         p['c2_b2'], p['c2_g'], p['c2_bt']))
    return _gin(h, A, eps[0, 2], p['c3_W1'], p['c3_b1'], p['c3_W2'],
                p['c3_b2'], p['c3_g'], p['c3_bt'])


def _att_pool(x, att_W):
    xa = _mm(x, att_W)
    gc = jnp.tanh(jnp.mean(xa, axis=0, keepdims=True))
    s = jax.nn.sigmoid(
        jax.lax.dot_general(x, gc, (((1,), (1,)), ((), ())),
                            preferred_element_type=jnp.float32,
                            precision=jax.lax.Precision.HIGHEST))
    return jnp.dot(jnp.transpose(s), x, preferred_element_type=jnp.float32,
                   precision=jax.lax.Precision.HIGHEST)


def _fused_kernel(f1_ref, f2_ref, hb_ref, A1_ref, A2_ref,
                  c1_W1, c1_b1, c1_W2, c1_b2, c1_g, c1_bt,
                  c2_W1, c2_b1, c2_W2, c2_b2, c2_g, c2_bt,
                  c3_W1, c3_b1, c3_W2, c3_b2, c3_g, c3_bt,
                  eps_ref, fc1_W, fc2_W, fc3_W, fc3_b, att_W,
                  tn_W, tn_WbT, tn_b, f1_W, f1_b, f2_W, f2_b, f3_W, f3_b,
                  sc_W, sc_b, out_pre, out_score):
    p = {
        'c1_W1': c1_W1[...], 'c1_b1': c1_b1[...], 'c1_W2': c1_W2[...],
        'c1_b2': c1_b2[...], 'c1_g': c1_g[...], 'c1_bt': c1_bt[...],
        'c2_W1': c2_W1[...], 'c2_b1': c2_b1[...], 'c2_W2': c2_W2[...],
        'c2_b2': c2_b2[...], 'c2_g': c2_g[...], 'c2_bt': c2_bt[...],
        'c3_W1': c3_W1[...], 'c3_b1': c3_b1[...], 'c3_W2': c3_W2[...],
        'c3_b2': c3_b2[...], 'c3_g': c3_g[...], 'c3_bt': c3_bt[...],
    }
    eps = eps_ref[...]

    A1 = A1_ref[...]
    A2 = A2_ref[...]
    h1 = _conv_pass(f1_ref[...], A1, eps, p)
    h2g = _conv_pass(f2_ref[...], A2, eps, p)

    pair = (h1[:, None, :] + h2g[None, :, :]).reshape(N * N, 32)
    e2d = jax.nn.relu(_mm(pair, fc1_W[...]))
    m2 = jax.nn.relu(_mm(e2d, fc2_W[...]))
    m3 = m2.reshape(N, N, 32)
    fc3row = fc3_W[...].reshape(1, 1, 32)
    energy = jnp.sum(m3 * fc3row, axis=2) + fc3_b[0, 0]

    emax = jnp.max(energy, axis=1, keepdims=True)
    ex = jnp.exp(energy - emax)
    att = ex / jnp.sum(ex, axis=1, keepdims=True)

    sim = jax.lax.dot_general(h2g, h1, (((1,), (1,)), ((), ())),
                              preferred_element_type=jnp.float32)
    cost = jnp.sum(att * sim)

    p1 = _att_pool(h1, att_W[...])
    p2 = _att_pool(h2g, att_W[...])

    e1c = jnp.transpose(p1).reshape(32, 1, 1)
    e2c = jnp.transpose(p2)
    S1 = jnp.sum(tn_W[...] * e1c, axis=0)
    sc16 = jnp.sum(S1 * e2c, axis=0, keepdims=True)
    comb = jnp.concatenate([p1, p2], axis=1)
    scores = jax.nn.relu(sc16 + _mm_hi(comb, tn_WbT[...]) + tn_b[...])
    scores = jax.nn.relu(_mm_hi(scores, f1_W[...]) + f1_b[...])
    scores = jax.nn.relu(_mm_hi(scores, f2_W[...]) + f2_b[...])
    scores = jax.nn.relu(_mm_hi(scores, f3_W[...]) + f3_b[...])
    bias = _mm_hi(scores, sc_W[...]) + sc_b[...]

    score = jax.nn.sigmoid(cost + bias)
    out_score[...] = score
    out_pre[...] = score * hb_ref[...]


def kernel(features_1, features_2, hb, edge_index_1, edge_index_2,
           c1_W1, c1_b1, c1_W2, c1_b2, c1_g, c1_bt,
           c2_W1, c2_b1, c2_W2, c2_b2, c2_g, c2_bt,
           c3_W1, c3_b1, c3_W2, c3_b2, c3_g, c3_bt,
           eps, fc1_W, fc2_W, fc3_W, fc3_b, att_W,
           tn_W, tn_Wb, tn_b,
           f1_W, f1_b, f2_W, f2_b, f3_W, f3_b,
           sc_W, sc_b):
    ei = jnp.stack([
        jnp.concatenate([edge_index_1[0], edge_index_1[1]]),
        jnp.concatenate([edge_index_2[0], edge_index_2[1]]),
    ]).astype(jnp.int32)
    A = _get_adjacency_sc()(ei)
    A1 = A[0].reshape(N, N)
    A2 = A[1].reshape(N, N)

    r = lambda v: v.reshape(1, -1)
    args = (
        features_1, features_2, hb.reshape(1, 1),
        A1, A2,
        c1_W1, r(c1_b1), c1_W2, r(c1_b2), r(c1_g), r(c1_bt),
        c2_W1, r(c2_b1), c2_W2, r(c2_b2), r(c2_g), r(c2_bt),
        c3_W1, r(c3_b1), c3_W2, r(c3_b2), r(c3_g), r(c3_bt),
        r(eps), fc1_W, fc2_W, fc3_W.reshape(1, 32), r(fc3_b), att_W,
        tn_W, tn_Wb.T, tn_b.reshape(1, 16),
        f1_W, r(f1_b), f2_W, r(f2_b), f3_W, r(f3_b),
        sc_W, r(sc_b),
    )
    out_pre, out_score = pl.pallas_call(
        _fused_kernel,
        out_shape=(jax.ShapeDtypeStruct((1, 1), jnp.float32),
                   jax.ShapeDtypeStruct((1, 1), jnp.float32)),
    )(*args)
    return (out_pre.reshape(-1), out_score.reshape(-1))
